# baseline (speedup 1.0000x reference)
import jax
import jax.numpy as jnp
from jax.experimental import pallas as pl
from jax.experimental.pallas import tpu as pltpu

_N, _D, _H, _O, _C = 10000, 128, 128, 2, 10
_RB = 400
_NB = _N // _RB


def _body(adj_ref, x_ref, W1_ref, b1_ref, W2_ref, b2_ref, muT_ref,
          z_ref, q_ref, u_ref, h_ref, p_ref):
    t = pl.program_id(0)
    i = pl.program_id(1)

    @pl.when(t == 0)
    def _pass1():
        @pl.when(i == 0)
        def _():
            u_ref[...] = jnp.dot(x_ref[...], W1_ref[...],
                                 preferred_element_type=jnp.float32)
        s = jnp.dot(adj_ref[...], u_ref[...],
                    preferred_element_type=jnp.float32)
        h_ref[pl.ds(i * _RB, _RB), :] = jnp.maximum(s + b1_ref[...], 0.0)

    @pl.when(t == 1)
    def _pass2():
        @pl.when(i == 0)
        def _():
            p_ref[...] = jnp.dot(h_ref[...], W2_ref[...],
                                 preferred_element_type=jnp.float32)
        z = jnp.dot(adj_ref[...], p_ref[...],
                    preferred_element_type=jnp.float32) + b2_ref[...]
        z_ref[...] = z
        d2 = ((z[:, 0:1] - muT_ref[0:1, :]) ** 2
              + (z[:, 1:2] - muT_ref[1:2, :]) ** 2)
        qr = 1.0 / (1.0 + d2 * 2.0 + 1e-6)
        qr = qr * jnp.sqrt(qr)
        q_ref[...] = qr / jnp.sum(qr, axis=1, keepdims=True)


def kernel(x, adj, W1, b1, W2, b2, mu):
    b1r = b1.reshape(1, _H)
    b2r = b2.reshape(1, _O)
    muT = mu.T
    grid = (2, _NB)
    z, q = pl.pallas_call(
        _body,
        grid=grid,
        in_specs=[
            pl.BlockSpec((_RB, _N), lambda t, i: (i, 0)),
            pl.BlockSpec((_N, _D), lambda t, i: (0, 0)),
            pl.BlockSpec((_D, _H), lambda t, i: (0, 0)),
            pl.BlockSpec((1, _H), lambda t, i: (0, 0)),
            pl.BlockSpec((_H, _O), lambda t, i: (0, 0)),
            pl.BlockSpec((1, _O), lambda t, i: (0, 0)),
            pl.BlockSpec((_O, _C), lambda t, i: (0, 0)),
        ],
        out_specs=[
            pl.BlockSpec((_RB, _O), lambda t, i: (i, 0)),
            pl.BlockSpec((_RB, _C), lambda t, i: (i, 0)),
        ],
        out_shape=[
            jax.ShapeDtypeStruct((_N, _O), jnp.float32),
            jax.ShapeDtypeStruct((_N, _C), jnp.float32),
        ],
        scratch_shapes=[
            pltpu.VMEM((_N, _D), jnp.float32),
            pltpu.VMEM((_N, _H), jnp.float32),
            pltpu.VMEM((_N, _O), jnp.float32),
        ],
    )(adj, x, W1, b1r, W2, b2r, muT)
    return (z, q)
